# baseline (speedup 1.0000x reference)
import functools
import math

import jax
import jax.numpy as jnp
from jax import lax
from jax.experimental import pallas as pl
from jax.experimental.pallas import tpu as pltpu
from jax.experimental.pallas import tpu_sc as plsc

CUTOFF = 10.0
LOG2 = math.log(2.0)

NW = 32
LANES = 16
CH = 1280


def _ssp(t):
    return jnp.maximum(t, 0.0) + jnp.log1p(jnp.exp(-jnp.abs(t))) - LOG2


def _ht_body(x_ref, aw1w_ref, aw1b_ref, l1w_ref, o_ref):
    t = lax.dot_general(x_ref[...], aw1w_ref[...], (((1,), (1,)), ((), ())),
                        preferred_element_type=jnp.float32) + aw1b_ref[...]
    o_ref[...] = lax.dot_general(l1w_ref[...], t, (((1,), (1,)), ((), ())),
                                 preferred_element_type=jnp.float32)


def _compute_ht(x, aw1_w, aw1_b, l1_w):
    n, h = x.shape
    return pl.pallas_call(
        _ht_body,
        out_shape=jax.ShapeDtypeStruct((h, n), jnp.float32),
    )(x, aw1_w, aw1_b.reshape(1, h), l1_w)


def _wt_body(ea_ref, ew_ref, d1w_ref, d1b_ref, d2e_ref, d2o_ref, b2e_ref,
             b2o_ref, o_ref):
    t = lax.dot_general(ea_ref[...], d1w_ref[...], (((1,), (1,)), ((), ())),
                        preferred_element_type=jnp.float32) + d1b_ref[...]
    t = _ssp(t)
    c = 0.5 * (jnp.cos(ew_ref[...] * (jnp.pi / CUTOFF)) + 1.0)
    we = (lax.dot_general(d2e_ref[...], t, (((1,), (1,)), ((), ())),
                          preferred_element_type=jnp.float32)
          + b2e_ref[...]) * c
    wo = (lax.dot_general(d2o_ref[...], t, (((1,), (1,)), ((), ())),
                          preferred_element_type=jnp.float32)
          + b2o_ref[...]) * c
    pe = lax.bitcast_convert_type(we.astype(jnp.bfloat16),
                                  jnp.uint16).astype(jnp.uint32)
    po = lax.bitcast_convert_type(wo.astype(jnp.bfloat16),
                                  jnp.uint16).astype(jnp.uint32)
    o_ref[...] = lax.bitcast_convert_type(pe | (po << 16), jnp.int32)


def _compute_wt(edge_attr, edge_weight, d1_w, d1_b, d2_w, d2_b, be):
    e, g = edge_attr.shape
    f = d1_w.shape[0]
    f2 = f // 2
    return pl.pallas_call(
        _wt_body,
        grid=(e // be,),
        in_specs=[
            pl.BlockSpec((be, g), lambda i: (i, 0)),
            pl.BlockSpec((1, be), lambda i: (0, i)),
            pl.BlockSpec((f, g), lambda i: (0, 0)),
            pl.BlockSpec((1, f), lambda i: (0, 0)),
            pl.BlockSpec((f2, f), lambda i: (0, 0)),
            pl.BlockSpec((f2, f), lambda i: (0, 0)),
            pl.BlockSpec((f2, 1), lambda i: (0, 0)),
            pl.BlockSpec((f2, 1), lambda i: (0, 0)),
        ],
        out_specs=pl.BlockSpec((f2, be), lambda i: (0, i)),
        out_shape=jax.ShapeDtypeStruct((f2, e), jnp.int32),
    )(edge_attr, edge_weight.reshape(1, e), d1_w, d1_b.reshape(1, f),
      d2_w[0::2], d2_w[1::2], d2_b[0::2].reshape(f2, 1),
      d2_b[1::2].reshape(f2, 1))


def _make_scatter(n, e, h):
    f_per = h // NW
    nchunks = e // CH
    mesh = plsc.VectorSubcoreMesh(core_axis_name="c", subcore_axis_name="s")

    @functools.partial(
        pl.kernel,
        out_type=jax.ShapeDtypeStruct((h * n,), jnp.float32),
        mesh=mesh,
        scratch_types=[
            pltpu.VMEM((f_per * n,), jnp.float32),
            pltpu.VMEM((f_per * n,), jnp.float32),
            pltpu.VMEM((2, CH), jnp.int32),
            pltpu.VMEM((2, 2, CH), jnp.int32),
            pltpu.SemaphoreType.DMA,
            pltpu.SemaphoreType.DMA,
        ],
        compiler_params=pltpu.CompilerParams(needs_layout_passes=False),
    )
    def scatter_kernel(ht_hbm, wt_hbm, pk_hbm, out_hbm,
                       h_v, a_v, pk_v, w_v, sem0, sem1):
        wid = lax.axis_index("s") * 2 + lax.axis_index("c")
        hoff = pl.multiple_of(wid * (f_per * n), 8)
        wbase = pl.multiple_of(wid * (2 * e), 8)
        sems = (sem0, sem1)

        def issue(ci, b):
            ci = lax.rem(ci + wid * 8, nchunks)
            e0 = pl.multiple_of(ci * CH, CH)
            pltpu.async_copy(pk_hbm.at[pl.ds(e0, CH)], pk_v.at[b], sems[b])
            for f2 in range(2):
                pltpu.async_copy(
                    w_flat_hbm_slice(wt_hbm, wbase + f2 * e + e0),
                    w_v.at[b, f2], sems[b])

        def w_flat_hbm_slice(ref, off):
            return ref.at[pl.ds(pl.multiple_of(off, 8), CH)]

        def drain(b):
            pltpu.make_async_copy(pk_hbm.at[pl.ds(0, CH)], pk_v.at[b],
                                  sems[b]).wait()
            for f2 in range(2):
                pltpu.make_async_copy(wt_hbm.at[pl.ds(0, CH)],
                                      w_v.at[b, f2], sems[b]).wait()

        def compute(b):
            @plsc.parallel_loop(0, CH, LANES, unroll=4)
            def _(off):
                pk = pk_v[b, pl.ds(off, LANES)]
                sidx = pk & 0x3FFF
                didx = (pk >> 14) & 0x3FFF
                for f2 in range(2):
                    wpk = w_v[b, f2, pl.ds(off, LANES)]
                    weven = plsc.bitcast(wpk << 16, jnp.float32)
                    wodd = plsc.bitcast(wpk & jnp.int32(-65536), jnp.float32)
                    for j, wrow in ((0, weven), (1, wodd)):
                        f = 2 * f2 + j
                        hrow = plsc.load_gather(h_v, [sidx + (f * n)])
                        plsc.addupdate_scatter(a_v, [didx + (f * n)],
                                               wrow * hrow)

        pltpu.sync_copy(ht_hbm.at[pl.ds(hoff, f_per * n)], h_v)

        @plsc.parallel_loop(0, f_per * n, LANES, unroll=8)
        def _(off):
            a_v[pl.ds(off, LANES)] = jnp.zeros((LANES,), jnp.float32)

        issue(0, 0)

        def pair_body(p, carry):
            ci = p * 2
            issue(ci + 1, 1)
            drain(0)
            compute(0)

            @pl.when(ci + 2 < nchunks)
            def _():
                issue(ci + 2, 0)

            drain(1)
            compute(1)
            return carry

        lax.fori_loop(0, nchunks // 2, pair_body, 0)
        pltpu.sync_copy(a_v, out_hbm.at[pl.ds(hoff, f_per * n)])

    return scatter_kernel


def _out_body(x_ref, at_ref, l2w_ref, l2b_ref, aw2w_ref, aw2b_ref, o_ref):
    conv = lax.dot_general(at_ref[...], l2w_ref[...], (((0,), (1,)), ((), ())),
                           preferred_element_type=jnp.float32) + l2b_ref[...]
    s = _ssp(conv)
    o_ref[...] = (lax.dot_general(s, aw2w_ref[...], (((1,), (1,)), ((), ())),
                                  preferred_element_type=jnp.float32)
                  + aw2b_ref[...] + x_ref[...])


def _compute_out(x, aggr_t, l2_w, l2_b, aw2_w, aw2_b):
    n, h = x.shape
    return pl.pallas_call(
        _out_body,
        out_shape=jax.ShapeDtypeStruct((n, h), jnp.float32),
    )(x, aggr_t, l2_w, l2_b.reshape(1, h), aw2_w, aw2_b.reshape(1, h))


def kernel(x, edge_index, edge_weight, edge_attr,
           aw1_w, aw1_b, d1_w, d1_b, d2_w, d2_b,
           l1_w, l2_w, l2_b, aw2_w, aw2_b):
    n, h = x.shape
    e = edge_attr.shape[0]
    src = edge_index[0].astype(jnp.int32)
    dst = edge_index[1].astype(jnp.int32)
    pk = src | (dst << 14)

    ht = _compute_ht(x, aw1_w, aw1_b, l1_w)
    wt = _compute_wt(edge_attr, edge_weight, d1_w, d1_b, d2_w, d2_b, be=6400)
    aggr_flat = _make_scatter(n, e, h)(
        ht.reshape(h * n), wt.reshape((h // 2) * e), pk)
    return _compute_out(x, aggr_flat.reshape(h, n), l2_w, l2_b, aw2_w, aw2_b)


# speedup vs baseline: 1.0092x; 1.0092x over previous
import functools
import math

import jax
import jax.numpy as jnp
from jax import lax
from jax.experimental import pallas as pl
from jax.experimental.pallas import tpu as pltpu
from jax.experimental.pallas import tpu_sc as plsc

CUTOFF = 10.0
LOG2 = math.log(2.0)

NW = 32
LANES = 16
CH = 640


def _ssp(t):
    return jnp.maximum(t, 0.0) + jnp.log1p(jnp.exp(-jnp.abs(t))) - LOG2


def _ht_body(x_ref, aw1w_ref, aw1b_ref, l1w_ref, o_ref):
    t = lax.dot_general(x_ref[...], aw1w_ref[...], (((1,), (1,)), ((), ())),
                        preferred_element_type=jnp.float32) + aw1b_ref[...]
    o_ref[...] = lax.dot_general(l1w_ref[...], t, (((1,), (1,)), ((), ())),
                                 preferred_element_type=jnp.float32)


def _compute_ht(x, aw1_w, aw1_b, l1_w):
    n, h = x.shape
    return pl.pallas_call(
        _ht_body,
        out_shape=jax.ShapeDtypeStruct((h, n), jnp.float32),
    )(x, aw1_w, aw1_b.reshape(1, h), l1_w)


def _wt_body(ea_ref, ew_ref, d1w_ref, d1b_ref, d2e_ref, d2o_ref, b2e_ref,
             b2o_ref, o_ref):
    t = lax.dot_general(ea_ref[...], d1w_ref[...], (((1,), (1,)), ((), ())),
                        preferred_element_type=jnp.float32) + d1b_ref[...]
    t = _ssp(t)
    c = 0.5 * (jnp.cos(ew_ref[...] * (jnp.pi / CUTOFF)) + 1.0)
    we = (lax.dot_general(d2e_ref[...], t, (((1,), (1,)), ((), ())),
                          preferred_element_type=jnp.float32)
          + b2e_ref[...]) * c
    wo = (lax.dot_general(d2o_ref[...], t, (((1,), (1,)), ((), ())),
                          preferred_element_type=jnp.float32)
          + b2o_ref[...]) * c
    pe = lax.bitcast_convert_type(we.astype(jnp.bfloat16),
                                  jnp.uint16).astype(jnp.uint32)
    po = lax.bitcast_convert_type(wo.astype(jnp.bfloat16),
                                  jnp.uint16).astype(jnp.uint32)
    o_ref[...] = lax.bitcast_convert_type(pe | (po << 16), jnp.int32)


def _compute_wt(edge_attr, edge_weight, d1_w, d1_b, d2_w, d2_b, be, i0, nb):
    e, g = edge_attr.shape
    f = d1_w.shape[0]
    f2 = f // 2
    return pl.pallas_call(
        _wt_body,
        grid=(nb,),
        in_specs=[
            pl.BlockSpec((be, g), lambda i: (i + i0, 0)),
            pl.BlockSpec((1, be), lambda i: (0, i + i0)),
            pl.BlockSpec((f, g), lambda i: (0, 0)),
            pl.BlockSpec((1, f), lambda i: (0, 0)),
            pl.BlockSpec((f2, f), lambda i: (0, 0)),
            pl.BlockSpec((f2, f), lambda i: (0, 0)),
            pl.BlockSpec((f2, 1), lambda i: (0, 0)),
            pl.BlockSpec((f2, 1), lambda i: (0, 0)),
        ],
        out_specs=pl.BlockSpec((f2, be), lambda i: (0, i)),
        out_shape=jax.ShapeDtypeStruct((f2, nb * be), jnp.int32),
    )(edge_attr, edge_weight.reshape(1, e), d1_w, d1_b.reshape(1, f),
      d2_w[0::2], d2_w[1::2], d2_b[0::2].reshape(f2, 1),
      d2_b[1::2].reshape(f2, 1))


def _make_scatter(n, e, h, ebase):
    f_per = h // NW
    nchunks = e // CH
    mesh = plsc.VectorSubcoreMesh(core_axis_name="c", subcore_axis_name="s")

    @functools.partial(
        pl.kernel,
        out_type=jax.ShapeDtypeStruct((h * n,), jnp.float32),
        mesh=mesh,
        scratch_types=[
            pltpu.VMEM((f_per * n,), jnp.float32),
            pltpu.VMEM((f_per * n,), jnp.float32),
            pltpu.VMEM((2, CH), jnp.int32),
            pltpu.VMEM((2, 2, CH), jnp.int32),
            pltpu.SemaphoreType.DMA,
            pltpu.SemaphoreType.DMA,
        ],
        compiler_params=pltpu.CompilerParams(needs_layout_passes=False),
    )
    def scatter_kernel(ht_hbm, wt_hbm, pk_hbm, out_hbm,
                       h_v, a_v, pk_v, w_v, sem0, sem1):
        wid = lax.axis_index("s") * 2 + lax.axis_index("c")
        hoff = pl.multiple_of(wid * (f_per * n), 8)
        wbase = pl.multiple_of(wid * (2 * e), 8)
        sems = (sem0, sem1)

        def issue(ci, b):
            ci = lax.rem(ci + wid * 8, nchunks)
            e0 = pl.multiple_of(ci * CH, CH)
            pltpu.async_copy(pk_hbm.at[pl.ds(e0 + ebase, CH)], pk_v.at[b],
                             sems[b])
            for f2 in range(2):
                pltpu.async_copy(
                    w_flat_hbm_slice(wt_hbm, wbase + f2 * e + e0),
                    w_v.at[b, f2], sems[b])

        def w_flat_hbm_slice(ref, off):
            return ref.at[pl.ds(pl.multiple_of(off, 8), CH)]

        def drain(b):
            pltpu.make_async_copy(pk_hbm.at[pl.ds(0, CH)], pk_v.at[b],
                                  sems[b]).wait()
            for f2 in range(2):
                pltpu.make_async_copy(wt_hbm.at[pl.ds(0, CH)],
                                      w_v.at[b, f2], sems[b]).wait()

        def compute(b):
            @plsc.parallel_loop(0, CH, LANES, unroll=4)
            def _(off):
                pk = pk_v[b, pl.ds(off, LANES)]
                sidx = pk & 0x3FFF
                didx = (pk >> 14) & 0x3FFF
                for f2 in range(2):
                    wpk = w_v[b, f2, pl.ds(off, LANES)]
                    weven = plsc.bitcast(wpk << 16, jnp.float32)
                    wodd = plsc.bitcast(wpk & jnp.int32(-65536), jnp.float32)
                    for j, wrow in ((0, weven), (1, wodd)):
                        f = 2 * f2 + j
                        hrow = plsc.load_gather(h_v, [sidx + (f * n)])
                        plsc.addupdate_scatter(a_v, [didx + (f * n)],
                                               wrow * hrow)

        pltpu.sync_copy(ht_hbm.at[pl.ds(hoff, f_per * n)], h_v)

        @plsc.parallel_loop(0, f_per * n, LANES, unroll=8)
        def _(off):
            a_v[pl.ds(off, LANES)] = jnp.zeros((LANES,), jnp.float32)

        issue(0, 0)

        def pair_body(p, carry):
            ci = p * 2
            issue(ci + 1, 1)
            drain(0)
            compute(0)

            @pl.when(ci + 2 < nchunks)
            def _():
                issue(ci + 2, 0)

            drain(1)
            compute(1)
            return carry

        lax.fori_loop(0, nchunks // 2, pair_body, 0)
        pltpu.sync_copy(a_v, out_hbm.at[pl.ds(hoff, f_per * n)])

    return scatter_kernel


def _out_body(x_ref, a0_ref, a1_ref, l2w_ref, l2b_ref, aw2w_ref, aw2b_ref,
              o_ref):
    at = a0_ref[...] + a1_ref[...]
    conv = lax.dot_general(at, l2w_ref[...], (((0,), (1,)), ((), ())),
                           preferred_element_type=jnp.float32) + l2b_ref[...]
    s = _ssp(conv)
    o_ref[...] = (lax.dot_general(s, aw2w_ref[...], (((1,), (1,)), ((), ())),
                                  preferred_element_type=jnp.float32)
                  + aw2b_ref[...] + x_ref[...])


def _compute_out(x, a0, a1, l2_w, l2_b, aw2_w, aw2_b):
    n, h = x.shape
    return pl.pallas_call(
        _out_body,
        out_shape=jax.ShapeDtypeStruct((n, h), jnp.float32),
    )(x, a0, a1, l2_w, l2_b.reshape(1, h), aw2_w, aw2_b.reshape(1, h))


def kernel(x, edge_index, edge_weight, edge_attr,
           aw1_w, aw1_b, d1_w, d1_b, d2_w, d2_b,
           l1_w, l2_w, l2_b, aw2_w, aw2_b):
    n, h = x.shape
    e = edge_attr.shape[0]
    src = edge_index[0].astype(jnp.int32)
    dst = edge_index[1].astype(jnp.int32)
    pk = src | (dst << 14)

    ht = _compute_ht(x, aw1_w, aw1_b, l1_w)
    ht_flat = ht.reshape(h * n)
    e2 = e // 2
    be = 6400
    aggr = []
    for hf in range(2):
        wt = _compute_wt(edge_attr, edge_weight, d1_w, d1_b, d2_w, d2_b,
                         be=be, i0=hf * (e2 // be), nb=e2 // be)
        aggr.append(_make_scatter(n, e2, h, ebase=hf * e2)(
            ht_flat, wt.reshape((h // 2) * e2), pk))
    return _compute_out(x, aggr[0].reshape(h, n), aggr[1].reshape(h, n),
                        l2_w, l2_b, aw2_w, aw2_b)


# speedup vs baseline: 1.1153x; 1.1052x over previous
import functools
import math

import jax
import jax.numpy as jnp
from jax import lax
from jax.experimental import pallas as pl
from jax.experimental.pallas import tpu as pltpu
from jax.experimental.pallas import tpu_sc as plsc

CUTOFF = 10.0
LOG2 = math.log(2.0)

NW = 32
LANES = 16
CH = 3200


def _ssp(t):
    return jnp.maximum(t, 0.0) + jnp.log1p(jnp.exp(-jnp.abs(t))) - LOG2


def _ht_body(x_ref, aw1w_ref, aw1b_ref, l1w_ref, o_ref):
    t = lax.dot_general(x_ref[...], aw1w_ref[...], (((1,), (1,)), ((), ())),
                        preferred_element_type=jnp.float32) + aw1b_ref[...]
    o_ref[...] = lax.dot_general(l1w_ref[...], t, (((1,), (1,)), ((), ())),
                                 preferred_element_type=jnp.float32)


def _compute_ht(x, aw1_w, aw1_b, l1_w):
    n, h = x.shape
    return pl.pallas_call(
        _ht_body,
        out_shape=jax.ShapeDtypeStruct((h, n), jnp.float32),
    )(x, aw1_w, aw1_b.reshape(1, h), l1_w)


def _wt_body(ea_ref, ew_ref, d1w_ref, d1b_ref, d2e_ref, d2o_ref, b2e_ref,
             b2o_ref, o_ref):
    t = lax.dot_general(ea_ref[...], d1w_ref[...], (((1,), (1,)), ((), ())),
                        preferred_element_type=jnp.float32) + d1b_ref[...]
    t = _ssp(t)
    c = 0.5 * (jnp.cos(ew_ref[...] * (jnp.pi / CUTOFF)) + 1.0)
    we = (lax.dot_general(d2e_ref[...], t, (((1,), (1,)), ((), ())),
                          preferred_element_type=jnp.float32)
          + b2e_ref[...]) * c
    wo = (lax.dot_general(d2o_ref[...], t, (((1,), (1,)), ((), ())),
                          preferred_element_type=jnp.float32)
          + b2o_ref[...]) * c
    pe = lax.bitcast_convert_type(we.astype(jnp.bfloat16),
                                  jnp.uint16).astype(jnp.uint32)
    po = lax.bitcast_convert_type(wo.astype(jnp.bfloat16),
                                  jnp.uint16).astype(jnp.uint32)
    o_ref[...] = lax.bitcast_convert_type(pe | (po << 16), jnp.int32)


def _compute_wt(edge_attr, edge_weight, d1_w, d1_b, d2_w, d2_b, be, i0, nb):
    e, g = edge_attr.shape
    f = d1_w.shape[0]
    f2 = f // 2
    return pl.pallas_call(
        _wt_body,
        grid=(nb,),
        in_specs=[
            pl.BlockSpec((be, g), lambda i: (i + i0, 0)),
            pl.BlockSpec((1, be), lambda i: (0, i + i0)),
            pl.BlockSpec((f, g), lambda i: (0, 0)),
            pl.BlockSpec((1, f), lambda i: (0, 0)),
            pl.BlockSpec((f2, f), lambda i: (0, 0)),
            pl.BlockSpec((f2, f), lambda i: (0, 0)),
            pl.BlockSpec((f2, 1), lambda i: (0, 0)),
            pl.BlockSpec((f2, 1), lambda i: (0, 0)),
        ],
        out_specs=pl.BlockSpec((f2, be), lambda i: (0, i)),
        out_shape=jax.ShapeDtypeStruct((f2, nb * be), jnp.int32),
    )(edge_attr, edge_weight.reshape(1, e), d1_w, d1_b.reshape(1, f),
      d2_w[0::2], d2_w[1::2], d2_b[0::2].reshape(f2, 1),
      d2_b[1::2].reshape(f2, 1))


def _make_scatter(n, e, h, ebase):
    f_per = h // NW
    nchunks = e // CH
    mesh = plsc.VectorSubcoreMesh(core_axis_name="c", subcore_axis_name="s")

    @functools.partial(
        pl.kernel,
        out_type=jax.ShapeDtypeStruct((h * n,), jnp.float32),
        mesh=mesh,
        scratch_types=[
            pltpu.VMEM((f_per * n,), jnp.float32),
            pltpu.VMEM((f_per * n,), jnp.float32),
            pltpu.VMEM((2, CH), jnp.int32),
            pltpu.VMEM((2, 2, CH), jnp.int32),
            pltpu.SemaphoreType.DMA,
            pltpu.SemaphoreType.DMA,
        ],
        compiler_params=pltpu.CompilerParams(needs_layout_passes=False),
    )
    def scatter_kernel(ht_hbm, wt_hbm, pk_hbm, out_hbm,
                       h_v, a_v, pk_v, w_v, sem0, sem1):
        wid = lax.axis_index("s") * 2 + lax.axis_index("c")
        hoff = pl.multiple_of(wid * (f_per * n), 8)
        wbase = pl.multiple_of(wid * (2 * e), 8)
        sems = (sem0, sem1)

        def issue(ci, b):
            ci = lax.rem(ci + wid * 8, nchunks)
            e0 = pl.multiple_of(ci * CH, CH)
            pltpu.async_copy(pk_hbm.at[pl.ds(e0 + ebase, CH)], pk_v.at[b],
                             sems[b])
            for f2 in range(2):
                pltpu.async_copy(
                    w_flat_hbm_slice(wt_hbm, wbase + f2 * e + e0),
                    w_v.at[b, f2], sems[b])

        def w_flat_hbm_slice(ref, off):
            return ref.at[pl.ds(pl.multiple_of(off, 8), CH)]

        def drain(b):
            pltpu.make_async_copy(pk_hbm.at[pl.ds(0, CH)], pk_v.at[b],
                                  sems[b]).wait()
            for f2 in range(2):
                pltpu.make_async_copy(wt_hbm.at[pl.ds(0, CH)],
                                      w_v.at[b, f2], sems[b]).wait()

        def compute(b):
            @plsc.parallel_loop(0, CH, LANES, unroll=4)
            def _(off):
                pk = pk_v[b, pl.ds(off, LANES)]
                sidx = pk & 0x3FFF
                didx = (pk >> 14) & 0x3FFF
                for f2 in range(2):
                    wpk = w_v[b, f2, pl.ds(off, LANES)]
                    weven = plsc.bitcast(wpk << 16, jnp.float32)
                    wodd = plsc.bitcast(wpk & jnp.int32(-65536), jnp.float32)
                    for j, wrow in ((0, weven), (1, wodd)):
                        f = 2 * f2 + j
                        hrow = plsc.load_gather(h_v, [sidx + (f * n)])
                        plsc.addupdate_scatter(a_v, [didx + (f * n)],
                                               wrow * hrow)

        issue(0, 0)
        pltpu.sync_copy(ht_hbm.at[pl.ds(hoff, f_per * n)], h_v)

        @plsc.parallel_loop(0, f_per * n, LANES, unroll=8)
        def _(off):
            a_v[pl.ds(off, LANES)] = jnp.zeros((LANES,), jnp.float32)

        def pair_body(p, carry):
            ci = p * 2
            issue(ci + 1, 1)
            drain(0)
            compute(0)

            @pl.when(ci + 2 < nchunks)
            def _():
                issue(ci + 2, 0)

            drain(1)
            compute(1)
            return carry

        lax.fori_loop(0, nchunks // 2, pair_body, 0)
        pltpu.sync_copy(a_v, out_hbm.at[pl.ds(hoff, f_per * n)])

    return scatter_kernel


def _out_body(x_ref, a0_ref, a1_ref, l2w_ref, l2b_ref, aw2w_ref, aw2b_ref,
              o_ref):
    at = a0_ref[...] + a1_ref[...]
    conv = lax.dot_general(at, l2w_ref[...], (((0,), (1,)), ((), ())),
                           preferred_element_type=jnp.float32) + l2b_ref[...]
    s = _ssp(conv)
    o_ref[...] = (lax.dot_general(s, aw2w_ref[...], (((1,), (1,)), ((), ())),
                                  preferred_element_type=jnp.float32)
                  + aw2b_ref[...] + x_ref[...])


def _compute_out(x, a0, a1, l2_w, l2_b, aw2_w, aw2_b):
    n, h = x.shape
    return pl.pallas_call(
        _out_body,
        out_shape=jax.ShapeDtypeStruct((n, h), jnp.float32),
    )(x, a0, a1, l2_w, l2_b.reshape(1, h), aw2_w, aw2_b.reshape(1, h))


def kernel(x, edge_index, edge_weight, edge_attr,
           aw1_w, aw1_b, d1_w, d1_b, d2_w, d2_b,
           l1_w, l2_w, l2_b, aw2_w, aw2_b):
    n, h = x.shape
    e = edge_attr.shape[0]
    src = edge_index[0].astype(jnp.int32)
    dst = edge_index[1].astype(jnp.int32)
    pk = src | (dst << 14)

    ht = _compute_ht(x, aw1_w, aw1_b, l1_w)
    ht_flat = ht.reshape(h * n)
    e2 = e // 2
    be = 6400
    aggr = []
    for hf in range(2):
        wt = _compute_wt(edge_attr, edge_weight, d1_w, d1_b, d2_w, d2_b,
                         be=be, i0=hf * (e2 // be), nb=e2 // be)
        aggr.append(_make_scatter(n, e2, h, ebase=hf * e2)(
            ht_flat, wt.reshape((h // 2) * e2), pk))
    return _compute_out(x, aggr[0].reshape(h, n), aggr[1].reshape(h, n),
                        l2_w, l2_b, aw2_w, aw2_b)


# speedup vs baseline: 1.3790x; 1.2364x over previous
import functools
import math

import jax
import jax.numpy as jnp
from jax import lax
from jax.experimental import pallas as pl
from jax.experimental.pallas import tpu as pltpu
from jax.experimental.pallas import tpu_sc as plsc

CUTOFF = 10.0
LOG2 = math.log(2.0)

NW = 32
LANES = 16
CH = 3200


def _ssp(t):
    return jnp.maximum(t, 0.0) + jnp.log1p(jnp.exp(-jnp.abs(t))) - LOG2


def _ht_body(x_ref, aw1w_ref, aw1b_ref, l1w_ref, o_ref):
    t = lax.dot_general(x_ref[...], aw1w_ref[...], (((1,), (1,)), ((), ())),
                        preferred_element_type=jnp.float32) + aw1b_ref[...]
    o_ref[...] = lax.dot_general(l1w_ref[...], t, (((1,), (1,)), ((), ())),
                                 preferred_element_type=jnp.float32)


def _compute_ht(x, aw1_w, aw1_b, l1_w):
    n, h = x.shape
    return pl.pallas_call(
        _ht_body,
        out_shape=jax.ShapeDtypeStruct((h, n), jnp.float32),
    )(x, aw1_w, aw1_b.reshape(1, h), l1_w)


def _wt_body(ea_ref, ew_ref, d1w_ref, d1b_ref, d2e_ref, d2o_ref, b2e_ref,
             b2o_ref, o_ref):
    t = lax.dot_general(ea_ref[...].astype(jnp.bfloat16), d1w_ref[...],
                        (((0,), (1,)), ((), ())),
                        preferred_element_type=jnp.float32) + d1b_ref[...]
    t = _ssp(t).astype(jnp.bfloat16)
    c = 0.5 * (jnp.cos(ew_ref[...] * (jnp.pi / CUTOFF)) + 1.0)
    we = (lax.dot_general(d2e_ref[...], t, (((1,), (1,)), ((), ())),
                          preferred_element_type=jnp.float32)
          + b2e_ref[...]) * c
    wo = (lax.dot_general(d2o_ref[...], t, (((1,), (1,)), ((), ())),
                          preferred_element_type=jnp.float32)
          + b2o_ref[...]) * c
    pe = lax.bitcast_convert_type(we.astype(jnp.bfloat16),
                                  jnp.uint16).astype(jnp.uint32)
    po = lax.bitcast_convert_type(wo.astype(jnp.bfloat16),
                                  jnp.uint16).astype(jnp.uint32)
    o_ref[...] = lax.bitcast_convert_type(pe | (po << 16), jnp.int32)


def _compute_wt(ea_t, edge_weight, d1_w, d1_b, d2_w, d2_b, be, i0, nb):
    g, e = ea_t.shape
    f = d1_w.shape[0]
    f2 = f // 2
    return pl.pallas_call(
        _wt_body,
        grid=(nb,),
        in_specs=[
            pl.BlockSpec((g, be), lambda i: (0, i + i0)),
            pl.BlockSpec((1, be), lambda i: (0, i + i0)),
            pl.BlockSpec((f, g), lambda i: (0, 0)),
            pl.BlockSpec((1, f), lambda i: (0, 0)),
            pl.BlockSpec((f2, f), lambda i: (0, 0)),
            pl.BlockSpec((f2, f), lambda i: (0, 0)),
            pl.BlockSpec((f2, 1), lambda i: (0, 0)),
            pl.BlockSpec((f2, 1), lambda i: (0, 0)),
        ],
        out_specs=pl.BlockSpec((f2, be), lambda i: (0, i)),
        out_shape=jax.ShapeDtypeStruct((f2, nb * be), jnp.int32),
    )(ea_t, edge_weight.reshape(1, e),
      d1_w.astype(jnp.bfloat16), d1_b.reshape(1, f),
      d2_w[0::2].astype(jnp.bfloat16), d2_w[1::2].astype(jnp.bfloat16),
      d2_b[0::2].reshape(f2, 1), d2_b[1::2].reshape(f2, 1))


def _make_scatter(n, e, h, ebase):
    f_per = h // NW
    nchunks = e // CH
    mesh = plsc.VectorSubcoreMesh(core_axis_name="c", subcore_axis_name="s")

    @functools.partial(
        pl.kernel,
        out_type=jax.ShapeDtypeStruct((h * n,), jnp.float32),
        mesh=mesh,
        scratch_types=[
            pltpu.VMEM((f_per * n,), jnp.float32),
            pltpu.VMEM((f_per * n,), jnp.float32),
            pltpu.VMEM((2, CH), jnp.int32),
            pltpu.VMEM((2, 2, CH), jnp.int32),
            pltpu.SemaphoreType.DMA,
            pltpu.SemaphoreType.DMA,
        ],
        compiler_params=pltpu.CompilerParams(needs_layout_passes=False),
    )
    def scatter_kernel(ht_hbm, wt_hbm, pk_hbm, out_hbm,
                       h_v, a_v, pk_v, w_v, sem0, sem1):
        wid = lax.axis_index("s") * 2 + lax.axis_index("c")
        hoff = pl.multiple_of(wid * (f_per * n), 8)
        wbase = pl.multiple_of(wid * (2 * e), 8)
        sems = (sem0, sem1)

        def issue(ci, b):
            ci = lax.rem(ci + wid * 8, nchunks)
            e0 = pl.multiple_of(ci * CH, CH)
            pltpu.async_copy(pk_hbm.at[pl.ds(e0 + ebase, CH)], pk_v.at[b],
                             sems[b])
            for f2 in range(2):
                pltpu.async_copy(
                    w_flat_hbm_slice(wt_hbm, wbase + f2 * e + e0),
                    w_v.at[b, f2], sems[b])

        def w_flat_hbm_slice(ref, off):
            return ref.at[pl.ds(pl.multiple_of(off, 8), CH)]

        def drain(b):
            pltpu.make_async_copy(pk_hbm.at[pl.ds(0, CH)], pk_v.at[b],
                                  sems[b]).wait()
            for f2 in range(2):
                pltpu.make_async_copy(wt_hbm.at[pl.ds(0, CH)],
                                      w_v.at[b, f2], sems[b]).wait()

        def compute(b):
            @plsc.parallel_loop(0, CH, LANES, unroll=4)
            def _(off):
                pk = pk_v[b, pl.ds(off, LANES)]
                sidx = pk & 0x3FFF
                didx = (pk >> 14) & 0x3FFF
                for f2 in range(2):
                    wpk = w_v[b, f2, pl.ds(off, LANES)]
                    weven = plsc.bitcast(wpk << 16, jnp.float32)
                    wodd = plsc.bitcast(wpk & jnp.int32(-65536), jnp.float32)
                    for j, wrow in ((0, weven), (1, wodd)):
                        f = 2 * f2 + j
                        hrow = plsc.load_gather(h_v, [sidx + (f * n)])
                        plsc.addupdate_scatter(a_v, [didx + (f * n)],
                                               wrow * hrow)

        issue(0, 0)
        pltpu.sync_copy(ht_hbm.at[pl.ds(hoff, f_per * n)], h_v)

        @plsc.parallel_loop(0, f_per * n, LANES, unroll=8)
        def _(off):
            a_v[pl.ds(off, LANES)] = jnp.zeros((LANES,), jnp.float32)

        def pair_body(p, carry):
            ci = p * 2
            issue(ci + 1, 1)
            drain(0)
            compute(0)

            @pl.when(ci + 2 < nchunks)
            def _():
                issue(ci + 2, 0)

            drain(1)
            compute(1)
            return carry

        lax.fori_loop(0, nchunks // 2, pair_body, 0)
        pltpu.sync_copy(a_v, out_hbm.at[pl.ds(hoff, f_per * n)])

    return scatter_kernel


def _out_body(x_ref, a0_ref, a1_ref, l2w_ref, l2b_ref, aw2w_ref, aw2b_ref,
              o_ref):
    at = a0_ref[...] + a1_ref[...]
    conv = lax.dot_general(at, l2w_ref[...], (((0,), (1,)), ((), ())),
                           preferred_element_type=jnp.float32) + l2b_ref[...]
    s = _ssp(conv)
    o_ref[...] = (lax.dot_general(s, aw2w_ref[...], (((1,), (1,)), ((), ())),
                                  preferred_element_type=jnp.float32)
                  + aw2b_ref[...] + x_ref[...])


def _compute_out(x, a0, a1, l2_w, l2_b, aw2_w, aw2_b):
    n, h = x.shape
    return pl.pallas_call(
        _out_body,
        out_shape=jax.ShapeDtypeStruct((n, h), jnp.float32),
    )(x, a0, a1, l2_w, l2_b.reshape(1, h), aw2_w, aw2_b.reshape(1, h))


def kernel(x, edge_index, edge_weight, edge_attr,
           aw1_w, aw1_b, d1_w, d1_b, d2_w, d2_b,
           l1_w, l2_w, l2_b, aw2_w, aw2_b):
    n, h = x.shape
    e = edge_attr.shape[0]
    src = edge_index[0].astype(jnp.int32)
    dst = edge_index[1].astype(jnp.int32)
    pk = src | (dst << 14)

    ht = _compute_ht(x, aw1_w, aw1_b, l1_w)
    ht_flat = ht.reshape(h * n)
    e2 = e // 2
    be = 16000
    aggr = []
    ea_t = edge_attr.T
    for hf in range(2):
        wt = _compute_wt(ea_t, edge_weight, d1_w, d1_b, d2_w, d2_b,
                         be=be, i0=hf * (e2 // be), nb=e2 // be)
        aggr.append(_make_scatter(n, e2, h, ebase=hf * e2)(
            ht_flat, wt.reshape((h // 2) * e2), pk))
    return _compute_out(x, aggr[0].reshape(h, n), aggr[1].reshape(h, n),
                        l2_w, l2_b, aw2_w, aw2_b)
